# baseline (speedup 1.0000x reference)
import functools

import jax
import jax.numpy as jnp
from jax import lax
from jax.experimental import pallas as pl
from jax.experimental.pallas import tpu as pltpu
from jax.experimental.pallas import tpu_sc as plsc

B, S, D = 1, 2048, 2048
H, KVH, HD = 16, 4, 128
E, TOPK = 8, 2
DFF_MOE, DFF_SH = 1024, 2048
EPS = 1e-05

BS = 512
BQ = 512
BK = 512
BLK = 128
NB = (2 * S + E * BLK) // BLK
PN = NB * BLK


def _rms_rows(x, w):
    v = jnp.mean(jnp.square(x), axis=-1, keepdims=True)
    return x * jax.lax.rsqrt(v + EPS) * w


def _pack2(x):
    c = x.shape[1] // 2
    hu = jax.lax.bitcast_convert_type(
        x[:, :c].astype(jnp.bfloat16), jnp.uint16).astype(jnp.uint32)
    lu = jax.lax.bitcast_convert_type(
        x[:, c:].astype(jnp.bfloat16), jnp.uint16).astype(jnp.uint32)
    return jax.lax.bitcast_convert_type((hu << 16) | lu, jnp.int32)


def _unpack2(w):
    wu = jax.lax.bitcast_convert_type(w, jnp.uint32)
    hf = jax.lax.bitcast_convert_type(
        (wu >> 16).astype(jnp.uint16), jnp.bfloat16).astype(jnp.float32)
    lf = jax.lax.bitcast_convert_type(
        (wu & 0xFFFF).astype(jnp.uint16), jnp.bfloat16).astype(jnp.float32)
    return jnp.concatenate([hf, lf], axis=1)


def _k1_body(x_ref, cos_ref, sin_ref, ln1_ref, qw_ref, kw_ref, vw_ref,
             qn_ref, kn_ref, q_out, k_out, v_out):
    x = x_ref[...]
    h = _rms_rows(x, ln1_ref[...])
    q = jnp.dot(h, qw_ref[...], preferred_element_type=jnp.float32)
    k = jnp.dot(h, kw_ref[...], preferred_element_type=jnp.float32)
    v = jnp.dot(h, vw_ref[...], preferred_element_type=jnp.float32)
    c = cos_ref[...]
    sn = sin_ref[...]
    qn = qn_ref[...]
    kn = kn_ref[...]
    half = HD // 2
    for hh in range(H):
        qh = q[:, hh * HD:(hh + 1) * HD]
        rot = jnp.concatenate([-qh[:, half:], qh[:, :half]], axis=1)
        qh = qh * c + rot * sn
        qh = _rms_rows(qh, qn)
        q_out[:, hh * HD:(hh + 1) * HD] = qh
    for hh in range(KVH):
        kh = k[:, hh * HD:(hh + 1) * HD]
        rot = jnp.concatenate([-kh[:, half:], kh[:, :half]], axis=1)
        kh = kh * c + rot * sn
        kh = _rms_rows(kh, kn)
        k_out[:, hh * HD:(hh + 1) * HD] = kh
    v_out[...] = v


def _k1(hs, cos_s, sin_s, ln1_w, q_w, k_w, v_w, qn_w, kn_w):
    return pl.pallas_call(
        _k1_body,
        grid=(S // BS,),
        in_specs=[
            pl.BlockSpec((BS, D), lambda i: (i, 0)),
            pl.BlockSpec((BS, HD), lambda i: (i, 0)),
            pl.BlockSpec((BS, HD), lambda i: (i, 0)),
            pl.BlockSpec((1, D), lambda i: (0, 0)),
            pl.BlockSpec((D, H * HD), lambda i: (0, 0)),
            pl.BlockSpec((D, KVH * HD), lambda i: (0, 0)),
            pl.BlockSpec((D, KVH * HD), lambda i: (0, 0)),
            pl.BlockSpec((1, HD), lambda i: (0, 0)),
            pl.BlockSpec((1, HD), lambda i: (0, 0)),
        ],
        out_specs=[
            pl.BlockSpec((BS, H * HD), lambda i: (i, 0)),
            pl.BlockSpec((BS, KVH * HD), lambda i: (i, 0)),
            pl.BlockSpec((BS, KVH * HD), lambda i: (i, 0)),
        ],
        out_shape=[
            jax.ShapeDtypeStruct((S, H * HD), jnp.float32),
            jax.ShapeDtypeStruct((S, KVH * HD), jnp.float32),
            jax.ShapeDtypeStruct((S, KVH * HD), jnp.float32),
        ],
    )(hs, cos_s, sin_s, ln1_w.reshape(1, D), q_w, k_w, v_w,
      qn_w.reshape(1, HD), kn_w.reshape(1, HD))


GQ = H // KVH


def _k23_body(q_ref, k_ref, v_ref, res_ref, ow_ref, ln2_ref, gw_ref,
              res2_out, npack_out, comb_out):
    i = pl.program_id(0)
    g = pl.program_id(1)
    nb = i + 1
    rows = i * BQ + jax.lax.broadcasted_iota(jnp.int32, (BQ, BK), 0)
    koff = pl.multiple_of(g * HD, HD)

    heads = []
    for j in range(GQ):
        q = q_ref[:, j * HD:(j + 1) * HD] * (HD ** -0.5)

        def body(kb, carry):
            acc, m, l = carry
            kblk = k_ref[pl.ds(kb * BK, BK), pl.ds(koff, HD)]
            vblk = v_ref[pl.ds(kb * BK, BK), pl.ds(koff, HD)]
            s = jnp.dot(q, kblk.T, preferred_element_type=jnp.float32)
            cols = kb * BK + jax.lax.broadcasted_iota(jnp.int32, (BQ, BK), 1)
            s = jnp.where(rows >= cols, s, -1e30)
            m_new = jnp.maximum(m, jnp.max(s, axis=1, keepdims=True))
            p = jnp.exp(s - m_new)
            scale = jnp.exp(m - m_new)
            l = l * scale + jnp.sum(p, axis=1, keepdims=True)
            acc = acc * scale + jnp.dot(p, vblk,
                                        preferred_element_type=jnp.float32)
            return acc, m_new, l

        acc0 = jnp.zeros((BQ, HD), jnp.float32)
        m0 = jnp.full((BQ, 1), -1e30, jnp.float32)
        l0 = jnp.zeros((BQ, 1), jnp.float32)
        acc, m, l = jax.lax.fori_loop(0, nb, body, (acc0, m0, l0))
        heads.append(acc / l)

    attn4 = jnp.concatenate(heads, axis=1)
    partial = jnp.dot(
        attn4, ow_ref[pl.ds(pl.multiple_of(g * GQ * HD, GQ * HD), GQ * HD), :],
        preferred_element_type=jnp.float32)

    @pl.when(g == 0)
    def _():
        res2_out[...] = partial

    @pl.when(g > 0)
    def _():
        res2_out[...] += partial

    @pl.when(g == KVH - 1)
    def _():
        res2 = res_ref[...] + res2_out[...]
        res2_out[...] = res2
        n = _rms_rows(res2, ln2_ref[...])
        npack_out[...] = _pack2(n)
        logits = jnp.dot(n, gw_ref[...], preferred_element_type=jnp.float32)
        lane = jax.lax.broadcasted_iota(jnp.int32, (BQ, 128), 1)
        valid = lane < E
        logits = jnp.where(valid, logits, -1e30)
        mx = jnp.max(logits, axis=1, keepdims=True)
        ex = jnp.exp(logits - mx)
        gates = ex / jnp.sum(ex, axis=1, keepdims=True)
        gates = jnp.where(valid, gates, -1.0)
        m1 = jnp.max(gates, axis=1, keepdims=True)
        i1 = jnp.min(jnp.where(gates == m1, lane, 999), axis=1, keepdims=True)
        g2 = jnp.where(lane == i1, -1.0, gates)
        m2 = jnp.max(g2, axis=1, keepdims=True)
        i2 = jnp.min(jnp.where(g2 == m2, lane, 999), axis=1, keepdims=True)
        tot = m1 + m2
        comb_out[...] = (jnp.where(lane == i1, m1 / tot, 0.0)
                         + jnp.where(lane == i2, m2 / tot, 0.0))


def _k23(q, k, v, res, o_w, ln2_w, gate_pad):
    return pl.pallas_call(
        _k23_body,
        grid=(S // BQ, KVH),
        in_specs=[
            pl.BlockSpec((BQ, GQ * HD), lambda i, g: (i, g)),
            pl.BlockSpec((S, KVH * HD), lambda i, g: (0, 0)),
            pl.BlockSpec((S, KVH * HD), lambda i, g: (0, 0)),
            pl.BlockSpec((BQ, D), lambda i, g: (i, 0)),
            pl.BlockSpec((H * HD, D), lambda i, g: (0, 0)),
            pl.BlockSpec((1, D), lambda i, g: (0, 0)),
            pl.BlockSpec((D, 128), lambda i, g: (0, 0)),
        ],
        out_specs=[
            pl.BlockSpec((BQ, D), lambda i, g: (i, 0)),
            pl.BlockSpec((BQ, D // 2), lambda i, g: (i, 0)),
            pl.BlockSpec((BQ, 128), lambda i, g: (i, 0)),
        ],
        out_shape=[
            jax.ShapeDtypeStruct((S, D), jnp.float32),
            jax.ShapeDtypeStruct((S, D // 2), jnp.int32),
            jax.ShapeDtypeStruct((S, 128), jnp.float32),
        ],
        compiler_params=pltpu.CompilerParams(
            vmem_limit_bytes=62 * 1024 * 1024),
    )(q, k, v, res, o_w, ln2_w.reshape(1, D), gate_pad)


def _k4_body(be_ref, xs_ref, eg_ref, eu_ref, ed_ref, out_ref):
    x = _unpack2(xs_ref[...])
    h1 = jnp.dot(x, eg_ref[0], preferred_element_type=jnp.float32)
    h2 = jnp.dot(x, eu_ref[0], preferred_element_type=jnp.float32)
    h = (h1 * jax.lax.logistic(h1)) * h2
    out_ref[...] = _pack2(jnp.dot(h, ed_ref[0],
                                  preferred_element_type=jnp.float32))


def _k4(xs, eg_w, eu_w, ed_w, block_expert):
    return pl.pallas_call(
        _k4_body,
        grid_spec=pltpu.PrefetchScalarGridSpec(
            num_scalar_prefetch=1,
            grid=(NB,),
            in_specs=[
                pl.BlockSpec((BLK, D // 2), lambda b, be: (b, 0)),
                pl.BlockSpec((1, D, DFF_MOE), lambda b, be: (be[b], 0, 0)),
                pl.BlockSpec((1, D, DFF_MOE), lambda b, be: (be[b], 0, 0)),
                pl.BlockSpec((1, DFF_MOE, D), lambda b, be: (be[b], 0, 0)),
            ],
            out_specs=pl.BlockSpec((BLK, D // 2), lambda b, be: (b, 0)),
        ),
        out_shape=jax.ShapeDtypeStruct((PN, D // 2), jnp.int32),
        compiler_params=pltpu.CompilerParams(
            vmem_limit_bytes=62 * 1024 * 1024),
    )(block_expert, xs, eg_w, eu_w, ed_w)


def _k5a_body(n_ref, sg_ref, su_ref, y_ref):
    n = _unpack2(n_ref[...])
    g = jnp.dot(n, sg_ref[...], preferred_element_type=jnp.float32)
    u = jnp.dot(n, su_ref[...], preferred_element_type=jnp.float32)
    y_ref[...] = _pack2((g * jax.lax.logistic(g)) * u)


def _k5a(normed, sg_w, su_w):
    return pl.pallas_call(
        _k5a_body,
        grid=(S // BS,),
        in_specs=[
            pl.BlockSpec((BS, D // 2), lambda i: (i, 0)),
            pl.BlockSpec((D, DFF_SH), lambda i: (0, 0)),
            pl.BlockSpec((D, DFF_SH), lambda i: (0, 0)),
        ],
        out_specs=pl.BlockSpec((BS, DFF_SH // 2), lambda i: (i, 0)),
        out_shape=jax.ShapeDtypeStruct((S, DFF_SH // 2), jnp.int32),
        compiler_params=pltpu.CompilerParams(
            vmem_limit_bytes=62 * 1024 * 1024),
    )(normed, sg_w, su_w)


def _k5b_body(y_ref, res2_ref, ea_ref, eb_ref, tw_ref, sd_ref, out_ref):
    sh = jnp.dot(_unpack2(y_ref[...]), sd_ref[...],
                 preferred_element_type=jnp.float32)
    routed = (tw_ref[:, 0:1] * _unpack2(ea_ref[...])
              + tw_ref[:, 1:2] * _unpack2(eb_ref[...]))
    out_ref[...] = res2_ref[...] + sh + routed


def _k5b(y, res2, ea, eb, twp, sd_w):
    return pl.pallas_call(
        _k5b_body,
        grid=(S // 256,),
        in_specs=[
            pl.BlockSpec((256, DFF_SH // 2), lambda i: (i, 0)),
            pl.BlockSpec((256, D), lambda i: (i, 0)),
            pl.BlockSpec((256, D // 2), lambda i: (i, 0)),
            pl.BlockSpec((256, D // 2), lambda i: (i, 0)),
            pl.BlockSpec((256, 128), lambda i: (i, 0)),
            pl.BlockSpec((DFF_SH, D), lambda i: (0, 0)),
        ],
        out_specs=pl.BlockSpec((256, D), lambda i: (i, 0)),
        out_shape=jax.ShapeDtypeStruct((S, D), jnp.float32),
        compiler_params=pltpu.CompilerParams(
            vmem_limit_bytes=62 * 1024 * 1024),
    )(y, res2, ea, eb, twp, sd_w)


SC_NC, SC_NS = 2, 16
SC_NW = SC_NC * SC_NS


def _sc_gather_i32(table, idx, n_rows, chunk):
    b_per_w = n_rows // SC_NW
    nchunk = b_per_w // chunk
    ncols = table.shape[1]
    mesh = plsc.VectorSubcoreMesh(core_axis_name="c", subcore_axis_name="s")

    @functools.partial(
        pl.kernel, mesh=mesh,
        out_type=jax.ShapeDtypeStruct((n_rows, ncols), jnp.int32),
        scratch_types=[
            pltpu.VMEM((chunk,), jnp.int32),
            pltpu.VMEM((chunk, ncols), jnp.int32),
            pltpu.SemaphoreType.DMA,
        ],
    )
    def k(table_hbm, idx_hbm, out_hbm, idx_v, rows_v, sem):
        wid = lax.axis_index("s") * SC_NC + lax.axis_index("c")
        base = wid * b_per_w
        for c in range(nchunk):
            off = base + c * chunk
            pltpu.sync_copy(idx_hbm.at[pl.ds(off, chunk)], idx_v)
            pltpu.async_copy(table_hbm.at[idx_v], rows_v, sem).wait()
            pltpu.sync_copy(rows_v, out_hbm.at[pl.ds(off, chunk)])

    return k(table, idx)


def kernel(hidden_states, position_ids, cos, sin, ln1_w, q_w, k_w, v_w, o_w,
           qn_w, kn_w, ln2_w, gate_w, sg_w, su_w, sd_w, eg_w, eu_w, ed_w):
    hs = hidden_states.reshape(S, D)
    cos_s = cos[:S]
    sin_s = sin[:S]

    q, k, v = _k1(hs, cos_s, sin_s, ln1_w, q_w, k_w, v_w, qn_w, kn_w)

    gate_pad = jnp.pad(gate_w, ((0, 0), (0, 128 - E)))
    res2, normed, comb = _k23(q, k, v, hs, o_w, ln2_w, gate_pad)

    cw = comb[:, :E]
    tw2, ti2 = jax.lax.top_k(cw, TOPK)
    e_flat = ti2.reshape(-1)
    onehot = (e_flat[:, None] == jnp.arange(E)[None, :]).astype(jnp.int32)
    ranks = jnp.cumsum(onehot, axis=0) - onehot
    rank = jnp.take_along_axis(ranks, e_flat[:, None], axis=1)[:, 0]
    gsz = jnp.sum(onehot, axis=0)
    gpad = ((gsz + BLK - 1) // BLK) * BLK
    off = jnp.concatenate([jnp.zeros((1,), jnp.int32),
                           jnp.cumsum(gpad)]).astype(jnp.int32)
    pos = off[e_flat] + rank
    tokp = jnp.zeros((PN,), jnp.int32).at[pos].set(
        jnp.arange(2 * S, dtype=jnp.int32) // 2)
    block_expert = jnp.clip(
        jnp.searchsorted(off[:E], jnp.arange(NB, dtype=jnp.int32) * BLK,
                         side='right') - 1, 0, E - 1).astype(jnp.int32)

    xs = _sc_gather_i32(normed, tokp, PN, 80)
    eout = _k4(xs, eg_w, eu_w, ed_w, block_expert)

    posr = pos.reshape(S, 2)
    gidx = jnp.concatenate([posr[:, 0], posr[:, 1]])
    gath = _sc_gather_i32(eout, gidx, 2 * S, 64)
    ea = gath[:S]
    eb = gath[S:]
    twp = jnp.pad(tw2, ((0, 0), (0, 128 - TOPK)))

    y = _k5a(normed, sg_w, su_w)
    out = _k5b(y, res2, ea, eb, twp, sd_w)
    return out.reshape(B, S, D)


# speedup vs baseline: 1.0199x; 1.0199x over previous
import functools

import jax
import jax.numpy as jnp
from jax import lax
from jax.experimental import pallas as pl
from jax.experimental.pallas import tpu as pltpu
from jax.experimental.pallas import tpu_sc as plsc

B, S, D = 1, 2048, 2048
H, KVH, HD = 16, 4, 128
E, TOPK = 8, 2
DFF_MOE, DFF_SH = 1024, 2048
EPS = 1e-05

BS = 512
BQ = 512
BK = 512
BLK = 128
NB = (2 * S + E * BLK) // BLK
PN = NB * BLK


def _rms_rows(x, w):
    v = jnp.mean(jnp.square(x), axis=-1, keepdims=True)
    return x * jax.lax.rsqrt(v + EPS) * w


def _pack2(x):
    c = x.shape[1] // 2
    hu = jax.lax.bitcast_convert_type(
        x[:, :c].astype(jnp.bfloat16), jnp.uint16).astype(jnp.uint32)
    lu = jax.lax.bitcast_convert_type(
        x[:, c:].astype(jnp.bfloat16), jnp.uint16).astype(jnp.uint32)
    return jax.lax.bitcast_convert_type((hu << 16) | lu, jnp.int32)


def _unpack2(w):
    wu = jax.lax.bitcast_convert_type(w, jnp.uint32)
    hf = jax.lax.bitcast_convert_type(
        (wu >> 16).astype(jnp.uint16), jnp.bfloat16).astype(jnp.float32)
    lf = jax.lax.bitcast_convert_type(
        (wu & 0xFFFF).astype(jnp.uint16), jnp.bfloat16).astype(jnp.float32)
    return jnp.concatenate([hf, lf], axis=1)


def _k1_body(x_ref, cos_ref, sin_ref, ln1_ref, qw_ref, kw_ref, vw_ref,
             qn_ref, kn_ref, q_out, k_out, v_out):
    x = x_ref[...]
    h = _rms_rows(x, ln1_ref[...])
    q = jnp.dot(h, qw_ref[...], preferred_element_type=jnp.float32)
    k = jnp.dot(h, kw_ref[...], preferred_element_type=jnp.float32)
    v = jnp.dot(h, vw_ref[...], preferred_element_type=jnp.float32)
    c = cos_ref[...]
    sn = sin_ref[...]
    qn = qn_ref[...]
    kn = kn_ref[...]
    half = HD // 2
    for hh in range(H):
        qh = q[:, hh * HD:(hh + 1) * HD]
        rot = jnp.concatenate([-qh[:, half:], qh[:, :half]], axis=1)
        qh = qh * c + rot * sn
        qh = _rms_rows(qh, qn)
        q_out[:, hh * HD:(hh + 1) * HD] = qh
    for hh in range(KVH):
        kh = k[:, hh * HD:(hh + 1) * HD]
        rot = jnp.concatenate([-kh[:, half:], kh[:, :half]], axis=1)
        kh = kh * c + rot * sn
        kh = _rms_rows(kh, kn)
        k_out[:, hh * HD:(hh + 1) * HD] = kh
    v_out[...] = v


def _k1(hs, cos_s, sin_s, ln1_w, q_w, k_w, v_w, qn_w, kn_w):
    return pl.pallas_call(
        _k1_body,
        grid=(S // BS,),
        in_specs=[
            pl.BlockSpec((BS, D), lambda i: (i, 0)),
            pl.BlockSpec((BS, HD), lambda i: (i, 0)),
            pl.BlockSpec((BS, HD), lambda i: (i, 0)),
            pl.BlockSpec((1, D), lambda i: (0, 0)),
            pl.BlockSpec((D, H * HD), lambda i: (0, 0)),
            pl.BlockSpec((D, KVH * HD), lambda i: (0, 0)),
            pl.BlockSpec((D, KVH * HD), lambda i: (0, 0)),
            pl.BlockSpec((1, HD), lambda i: (0, 0)),
            pl.BlockSpec((1, HD), lambda i: (0, 0)),
        ],
        out_specs=[
            pl.BlockSpec((BS, H * HD), lambda i: (i, 0)),
            pl.BlockSpec((BS, KVH * HD), lambda i: (i, 0)),
            pl.BlockSpec((BS, KVH * HD), lambda i: (i, 0)),
        ],
        out_shape=[
            jax.ShapeDtypeStruct((S, H * HD), jnp.float32),
            jax.ShapeDtypeStruct((S, KVH * HD), jnp.float32),
            jax.ShapeDtypeStruct((S, KVH * HD), jnp.float32),
        ],
    )(hs, cos_s, sin_s, ln1_w.reshape(1, D), q_w, k_w, v_w,
      qn_w.reshape(1, HD), kn_w.reshape(1, HD))


GQ = H // KVH


def _k2_body(q_ref, k_ref, v_ref, o_ref):
    i = pl.program_id(1)
    nb = i + 1
    rows = i * BQ + jax.lax.broadcasted_iota(jnp.int32, (BQ, BK), 0)

    for j in range(GQ):
        q = q_ref[:, j * HD:(j + 1) * HD] * (HD ** -0.5)

        def body(kb, carry):
            acc, m, l = carry
            kblk = k_ref[pl.ds(kb * BK, BK), :]
            vblk = v_ref[pl.ds(kb * BK, BK), :]
            s = jnp.dot(q, kblk.T, preferred_element_type=jnp.float32)
            cols = kb * BK + jax.lax.broadcasted_iota(jnp.int32, (BQ, BK), 1)
            s = jnp.where(rows >= cols, s, -1e30)
            m_new = jnp.maximum(m, jnp.max(s, axis=1, keepdims=True))
            p = jnp.exp(s - m_new)
            scale = jnp.exp(m - m_new)
            l = l * scale + jnp.sum(p, axis=1, keepdims=True)
            acc = acc * scale + jnp.dot(p, vblk,
                                        preferred_element_type=jnp.float32)
            return acc, m_new, l

        acc0 = jnp.zeros((BQ, HD), jnp.float32)
        m0 = jnp.full((BQ, 1), -1e30, jnp.float32)
        l0 = jnp.zeros((BQ, 1), jnp.float32)
        acc, m, l = jax.lax.fori_loop(0, nb, body, (acc0, m0, l0))
        o_ref[:, j * HD:(j + 1) * HD] = acc / l


def _k2(q, k, v):
    return pl.pallas_call(
        _k2_body,
        grid=(KVH, S // BQ),
        in_specs=[
            pl.BlockSpec((BQ, GQ * HD), lambda g, i: (i, g)),
            pl.BlockSpec((S, HD), lambda g, i: (0, g)),
            pl.BlockSpec((S, HD), lambda g, i: (0, g)),
        ],
        out_specs=pl.BlockSpec((BQ, GQ * HD), lambda g, i: (i, g)),
        out_shape=jax.ShapeDtypeStruct((S, H * HD), jnp.float32),
    )(q, k, v)


def _k3_body(attn_ref, res_ref, ow_ref, ln2_ref, gw_ref,
             res2_out, normed_out, comb_out):
    a = jnp.dot(attn_ref[...], ow_ref[...], preferred_element_type=jnp.float32)
    res2 = res_ref[...] + a
    res2_out[...] = res2
    n = _rms_rows(res2, ln2_ref[...])
    normed_out[...] = _pack2(n)
    logits = jnp.dot(n, gw_ref[...], preferred_element_type=jnp.float32)
    lane = jax.lax.broadcasted_iota(jnp.int32, (BS, 128), 1)
    valid = lane < E
    logits = jnp.where(valid, logits, -1e30)
    mx = jnp.max(logits, axis=1, keepdims=True)
    ex = jnp.exp(logits - mx)
    gates = ex / jnp.sum(ex, axis=1, keepdims=True)
    gates = jnp.where(valid, gates, -1.0)
    m1 = jnp.max(gates, axis=1, keepdims=True)
    i1 = jnp.min(jnp.where(gates == m1, lane, 999), axis=1, keepdims=True)
    g2 = jnp.where(lane == i1, -1.0, gates)
    m2 = jnp.max(g2, axis=1, keepdims=True)
    i2 = jnp.min(jnp.where(g2 == m2, lane, 999), axis=1, keepdims=True)
    tot = m1 + m2
    comb = jnp.where(lane == i1, m1 / tot, 0.0) + jnp.where(lane == i2,
                                                            m2 / tot, 0.0)
    comb_out[...] = comb


def _k3(attn, res, o_w, ln2_w, gate_pad):
    return pl.pallas_call(
        _k3_body,
        grid=(S // BS,),
        in_specs=[
            pl.BlockSpec((BS, H * HD), lambda i: (i, 0)),
            pl.BlockSpec((BS, D), lambda i: (i, 0)),
            pl.BlockSpec((H * HD, D), lambda i: (0, 0)),
            pl.BlockSpec((1, D), lambda i: (0, 0)),
            pl.BlockSpec((D, 128), lambda i: (0, 0)),
        ],
        out_specs=[
            pl.BlockSpec((BS, D), lambda i: (i, 0)),
            pl.BlockSpec((BS, D // 2), lambda i: (i, 0)),
            pl.BlockSpec((BS, 128), lambda i: (i, 0)),
        ],
        out_shape=[
            jax.ShapeDtypeStruct((S, D), jnp.float32),
            jax.ShapeDtypeStruct((S, D // 2), jnp.int32),
            jax.ShapeDtypeStruct((S, 128), jnp.float32),
        ],
    )(attn, res, o_w, ln2_w.reshape(1, D), gate_pad)


def _k4_body(be_ref, xs_ref, eg_ref, eu_ref, ed_ref, out_ref):
    x = _unpack2(xs_ref[...])
    h1 = jnp.dot(x, eg_ref[0], preferred_element_type=jnp.float32)
    h2 = jnp.dot(x, eu_ref[0], preferred_element_type=jnp.float32)
    h = (h1 * jax.lax.logistic(h1)) * h2
    out_ref[...] = _pack2(jnp.dot(h, ed_ref[0],
                                  preferred_element_type=jnp.float32))


def _k4(xs, eg_w, eu_w, ed_w, block_expert):
    return pl.pallas_call(
        _k4_body,
        grid_spec=pltpu.PrefetchScalarGridSpec(
            num_scalar_prefetch=1,
            grid=(NB,),
            in_specs=[
                pl.BlockSpec((BLK, D // 2), lambda b, be: (b, 0)),
                pl.BlockSpec((1, D, DFF_MOE), lambda b, be: (be[b], 0, 0)),
                pl.BlockSpec((1, D, DFF_MOE), lambda b, be: (be[b], 0, 0)),
                pl.BlockSpec((1, DFF_MOE, D), lambda b, be: (be[b], 0, 0)),
            ],
            out_specs=pl.BlockSpec((BLK, D // 2), lambda b, be: (b, 0)),
        ),
        out_shape=jax.ShapeDtypeStruct((PN, D // 2), jnp.int32),
        compiler_params=pltpu.CompilerParams(
            vmem_limit_bytes=62 * 1024 * 1024),
    )(block_expert, xs, eg_w, eu_w, ed_w)


def _k5a_body(n_ref, sg_ref, su_ref, y_ref):
    n = _unpack2(n_ref[...])
    g = jnp.dot(n, sg_ref[...], preferred_element_type=jnp.float32)
    u = jnp.dot(n, su_ref[...], preferred_element_type=jnp.float32)
    y_ref[...] = _pack2((g * jax.lax.logistic(g)) * u)


def _k5a(normed, sg_w, su_w):
    return pl.pallas_call(
        _k5a_body,
        grid=(S // BS,),
        in_specs=[
            pl.BlockSpec((BS, D // 2), lambda i: (i, 0)),
            pl.BlockSpec((D, DFF_SH), lambda i: (0, 0)),
            pl.BlockSpec((D, DFF_SH), lambda i: (0, 0)),
        ],
        out_specs=pl.BlockSpec((BS, DFF_SH // 2), lambda i: (i, 0)),
        out_shape=jax.ShapeDtypeStruct((S, DFF_SH // 2), jnp.int32),
        compiler_params=pltpu.CompilerParams(
            vmem_limit_bytes=62 * 1024 * 1024),
    )(normed, sg_w, su_w)


def _k5b_body(y_ref, res2_ref, ea_ref, eb_ref, tw_ref, sd_ref, out_ref):
    sh = jnp.dot(_unpack2(y_ref[...]), sd_ref[...],
                 preferred_element_type=jnp.float32)
    routed = (tw_ref[:, 0:1] * _unpack2(ea_ref[...])
              + tw_ref[:, 1:2] * _unpack2(eb_ref[...]))
    out_ref[...] = res2_ref[...] + sh + routed


def _k5b(y, res2, ea, eb, twp, sd_w):
    return pl.pallas_call(
        _k5b_body,
        grid=(S // 256,),
        in_specs=[
            pl.BlockSpec((256, DFF_SH // 2), lambda i: (i, 0)),
            pl.BlockSpec((256, D), lambda i: (i, 0)),
            pl.BlockSpec((256, D // 2), lambda i: (i, 0)),
            pl.BlockSpec((256, D // 2), lambda i: (i, 0)),
            pl.BlockSpec((256, 128), lambda i: (i, 0)),
            pl.BlockSpec((DFF_SH, D), lambda i: (0, 0)),
        ],
        out_specs=pl.BlockSpec((256, D), lambda i: (i, 0)),
        out_shape=jax.ShapeDtypeStruct((S, D), jnp.float32),
        compiler_params=pltpu.CompilerParams(
            vmem_limit_bytes=62 * 1024 * 1024),
    )(y, res2, ea, eb, twp, sd_w)


SC_NC, SC_NS = 2, 16
SC_NW = SC_NC * SC_NS


def _sc_gather_i32(table, idx, n_rows, chunk):
    b_per_w = n_rows // SC_NW
    nchunk = b_per_w // chunk
    ncols = table.shape[1]
    mesh = plsc.VectorSubcoreMesh(core_axis_name="c", subcore_axis_name="s")

    @functools.partial(
        pl.kernel, mesh=mesh,
        out_type=jax.ShapeDtypeStruct((n_rows, ncols), jnp.int32),
        scratch_types=[
            pltpu.VMEM((chunk,), jnp.int32),
            pltpu.VMEM((chunk, ncols), jnp.int32),
            pltpu.SemaphoreType.DMA,
        ],
    )
    def k(table_hbm, idx_hbm, out_hbm, idx_v, rows_v, sem):
        wid = lax.axis_index("s") * SC_NC + lax.axis_index("c")
        base = wid * b_per_w
        for c in range(nchunk):
            off = base + c * chunk
            pltpu.sync_copy(idx_hbm.at[pl.ds(off, chunk)], idx_v)
            pltpu.async_copy(table_hbm.at[idx_v], rows_v, sem).wait()
            pltpu.sync_copy(rows_v, out_hbm.at[pl.ds(off, chunk)])

    return k(table, idx)


def kernel(hidden_states, position_ids, cos, sin, ln1_w, q_w, k_w, v_w, o_w,
           qn_w, kn_w, ln2_w, gate_w, sg_w, su_w, sd_w, eg_w, eu_w, ed_w):
    hs = hidden_states.reshape(S, D)
    cos_s = cos[:S]
    sin_s = sin[:S]

    q, k, v = _k1(hs, cos_s, sin_s, ln1_w, q_w, k_w, v_w, qn_w, kn_w)
    attn = _k2(q, k, v)

    gate_pad = jnp.pad(gate_w, ((0, 0), (0, 128 - E)))
    res2, normed, comb = _k3(attn, hs, o_w, ln2_w, gate_pad)

    cw = comb[:, :E]
    tw2, ti2 = jax.lax.top_k(cw, TOPK)
    e_flat = ti2.reshape(-1)
    onehot = (e_flat[:, None] == jnp.arange(E)[None, :]).astype(jnp.int32)
    ranks = jnp.cumsum(onehot, axis=0) - onehot
    rank = jnp.take_along_axis(ranks, e_flat[:, None], axis=1)[:, 0]
    gsz = jnp.sum(onehot, axis=0)
    gpad = ((gsz + BLK - 1) // BLK) * BLK
    off = jnp.concatenate([jnp.zeros((1,), jnp.int32),
                           jnp.cumsum(gpad)]).astype(jnp.int32)
    pos = off[e_flat] + rank
    tokp = jnp.zeros((PN,), jnp.int32).at[pos].set(
        jnp.arange(2 * S, dtype=jnp.int32) // 2)
    block_expert = jnp.clip(
        jnp.searchsorted(off[:E], jnp.arange(NB, dtype=jnp.int32) * BLK,
                         side='right') - 1, 0, E - 1).astype(jnp.int32)

    xs = _sc_gather_i32(normed, tokp, PN, 80)
    eout = _k4(xs, eg_w, eu_w, ed_w, block_expert)

    posr = pos.reshape(S, 2)
    gidx = jnp.concatenate([posr[:, 0], posr[:, 1]])
    gath = _sc_gather_i32(eout, gidx, 2 * S, 64)
    ea = gath[:S]
    eb = gath[S:]
    twp = jnp.pad(tw2, ((0, 0), (0, 128 - TOPK)))

    y = _k5a(normed, sg_w, su_w)
    out = _k5b(y, res2, ea, eb, twp, sd_w)
    return out.reshape(B, S, D)


# speedup vs baseline: 1.0210x; 1.0010x over previous
import functools

import jax
import jax.numpy as jnp
from jax import lax
from jax.experimental import pallas as pl
from jax.experimental.pallas import tpu as pltpu
from jax.experimental.pallas import tpu_sc as plsc

B, S, D = 1, 2048, 2048
H, KVH, HD = 16, 4, 128
E, TOPK = 8, 2
DFF_MOE, DFF_SH = 1024, 2048
EPS = 1e-05

BS = 512
BQ = 512
BK = 512
BLK = 128
NB = (2 * S + E * BLK) // BLK
PN = NB * BLK


def _rms_rows(x, w):
    v = jnp.mean(jnp.square(x), axis=-1, keepdims=True)
    return x * jax.lax.rsqrt(v + EPS) * w


def _pack2(x):
    c = x.shape[1] // 2
    hu = jax.lax.bitcast_convert_type(
        x[:, :c].astype(jnp.bfloat16), jnp.uint16).astype(jnp.uint32)
    lu = jax.lax.bitcast_convert_type(
        x[:, c:].astype(jnp.bfloat16), jnp.uint16).astype(jnp.uint32)
    return jax.lax.bitcast_convert_type((hu << 16) | lu, jnp.int32)


def _unpack2(w):
    wu = jax.lax.bitcast_convert_type(w, jnp.uint32)
    hf = jax.lax.bitcast_convert_type(
        (wu >> 16).astype(jnp.uint16), jnp.bfloat16).astype(jnp.float32)
    lf = jax.lax.bitcast_convert_type(
        (wu & 0xFFFF).astype(jnp.uint16), jnp.bfloat16).astype(jnp.float32)
    return jnp.concatenate([hf, lf], axis=1)


def _k1_body(x_ref, cos_ref, sin_ref, ln1_ref, qw_ref, kw_ref, vw_ref,
             qn_ref, kn_ref, q_out, k_out, v_out):
    x = x_ref[...]
    h = _rms_rows(x, ln1_ref[...])
    q = jnp.dot(h, qw_ref[...], preferred_element_type=jnp.float32)
    k = jnp.dot(h, kw_ref[...], preferred_element_type=jnp.float32)
    v = jnp.dot(h, vw_ref[...], preferred_element_type=jnp.float32)
    c = cos_ref[...]
    sn = sin_ref[...]
    qn = qn_ref[...]
    kn = kn_ref[...]
    half = HD // 2
    for hh in range(H):
        qh = q[:, hh * HD:(hh + 1) * HD]
        rot = jnp.concatenate([-qh[:, half:], qh[:, :half]], axis=1)
        qh = qh * c + rot * sn
        qh = _rms_rows(qh, qn)
        q_out[:, hh * HD:(hh + 1) * HD] = qh
    for hh in range(KVH):
        kh = k[:, hh * HD:(hh + 1) * HD]
        rot = jnp.concatenate([-kh[:, half:], kh[:, :half]], axis=1)
        kh = kh * c + rot * sn
        kh = _rms_rows(kh, kn)
        k_out[:, hh * HD:(hh + 1) * HD] = kh
    v_out[...] = v


def _k1(hs, cos_s, sin_s, ln1_w, q_w, k_w, v_w, qn_w, kn_w):
    return pl.pallas_call(
        _k1_body,
        grid=(S // BS,),
        in_specs=[
            pl.BlockSpec((BS, D), lambda i: (i, 0)),
            pl.BlockSpec((BS, HD), lambda i: (i, 0)),
            pl.BlockSpec((BS, HD), lambda i: (i, 0)),
            pl.BlockSpec((1, D), lambda i: (0, 0)),
            pl.BlockSpec((D, H * HD), lambda i: (0, 0)),
            pl.BlockSpec((D, KVH * HD), lambda i: (0, 0)),
            pl.BlockSpec((D, KVH * HD), lambda i: (0, 0)),
            pl.BlockSpec((1, HD), lambda i: (0, 0)),
            pl.BlockSpec((1, HD), lambda i: (0, 0)),
        ],
        out_specs=[
            pl.BlockSpec((BS, H * HD), lambda i: (i, 0)),
            pl.BlockSpec((BS, KVH * HD), lambda i: (i, 0)),
            pl.BlockSpec((BS, KVH * HD), lambda i: (i, 0)),
        ],
        out_shape=[
            jax.ShapeDtypeStruct((S, H * HD), jnp.float32),
            jax.ShapeDtypeStruct((S, KVH * HD), jnp.float32),
            jax.ShapeDtypeStruct((S, KVH * HD), jnp.float32),
        ],
    )(hs, cos_s, sin_s, ln1_w.reshape(1, D), q_w, k_w, v_w,
      qn_w.reshape(1, HD), kn_w.reshape(1, HD))


GQ = H // KVH


def _k2_body(q_ref, k_ref, v_ref, o_ref):
    i = pl.program_id(1)
    nb = i + 1
    rows = i * BQ + jax.lax.broadcasted_iota(jnp.int32, (BQ, BK), 0)

    for j in range(GQ):
        q = q_ref[:, j * HD:(j + 1) * HD] * (HD ** -0.5)

        def body(kb, carry):
            acc, m, l = carry
            kblk = k_ref[pl.ds(kb * BK, BK), :]
            vblk = v_ref[pl.ds(kb * BK, BK), :]
            s = jnp.dot(q, kblk.T, preferred_element_type=jnp.float32)
            cols = kb * BK + jax.lax.broadcasted_iota(jnp.int32, (BQ, BK), 1)
            s = jnp.where(rows >= cols, s, -1e30)
            m_new = jnp.maximum(m, jnp.max(s, axis=1, keepdims=True))
            p = jnp.exp(s - m_new)
            scale = jnp.exp(m - m_new)
            l = l * scale + jnp.sum(p, axis=1, keepdims=True)
            acc = acc * scale + jnp.dot(p, vblk,
                                        preferred_element_type=jnp.float32)
            return acc, m_new, l

        acc0 = jnp.zeros((BQ, HD), jnp.float32)
        m0 = jnp.full((BQ, 1), -1e30, jnp.float32)
        l0 = jnp.zeros((BQ, 1), jnp.float32)
        acc, m, l = jax.lax.fori_loop(0, nb, body, (acc0, m0, l0))
        o_ref[:, j * HD:(j + 1) * HD] = acc / l


def _k2(q, k, v):
    return pl.pallas_call(
        _k2_body,
        grid=(KVH, S // BQ),
        in_specs=[
            pl.BlockSpec((BQ, GQ * HD), lambda g, i: (i, g)),
            pl.BlockSpec((S, HD), lambda g, i: (0, g)),
            pl.BlockSpec((S, HD), lambda g, i: (0, g)),
        ],
        out_specs=pl.BlockSpec((BQ, GQ * HD), lambda g, i: (i, g)),
        out_shape=jax.ShapeDtypeStruct((S, H * HD), jnp.float32),
    )(q, k, v)


def _k3_body(attn_ref, res_ref, ow_ref, ln2_ref, gw_ref,
             res2_out, normed_out, comb_out):
    a = jnp.dot(attn_ref[...], ow_ref[...], preferred_element_type=jnp.float32)
    res2 = res_ref[...] + a
    res2_out[...] = res2
    n = _rms_rows(res2, ln2_ref[...])
    normed_out[...] = _pack2(n)
    logits = jnp.dot(n, gw_ref[...], preferred_element_type=jnp.float32)
    lane = jax.lax.broadcasted_iota(jnp.int32, (BS, 128), 1)
    valid = lane < E
    logits = jnp.where(valid, logits, -1e30)
    mx = jnp.max(logits, axis=1, keepdims=True)
    ex = jnp.exp(logits - mx)
    gates = ex / jnp.sum(ex, axis=1, keepdims=True)
    gates = jnp.where(valid, gates, -1.0)
    m1 = jnp.max(gates, axis=1, keepdims=True)
    i1 = jnp.min(jnp.where(gates == m1, lane, 999), axis=1, keepdims=True)
    g2 = jnp.where(lane == i1, -1.0, gates)
    m2 = jnp.max(g2, axis=1, keepdims=True)
    i2 = jnp.min(jnp.where(g2 == m2, lane, 999), axis=1, keepdims=True)
    tot = m1 + m2
    comb = jnp.where(lane == i1, m1 / tot, 0.0) + jnp.where(lane == i2,
                                                            m2 / tot, 0.0)
    comb_out[...] = comb


def _k3(attn, res, o_w, ln2_w, gate_pad):
    return pl.pallas_call(
        _k3_body,
        grid=(S // BS,),
        in_specs=[
            pl.BlockSpec((BS, H * HD), lambda i: (i, 0)),
            pl.BlockSpec((BS, D), lambda i: (i, 0)),
            pl.BlockSpec((H * HD, D), lambda i: (0, 0)),
            pl.BlockSpec((1, D), lambda i: (0, 0)),
            pl.BlockSpec((D, 128), lambda i: (0, 0)),
        ],
        out_specs=[
            pl.BlockSpec((BS, D), lambda i: (i, 0)),
            pl.BlockSpec((BS, D // 2), lambda i: (i, 0)),
            pl.BlockSpec((BS, 128), lambda i: (i, 0)),
        ],
        out_shape=[
            jax.ShapeDtypeStruct((S, D), jnp.float32),
            jax.ShapeDtypeStruct((S, D // 2), jnp.int32),
            jax.ShapeDtypeStruct((S, 128), jnp.float32),
        ],
    )(attn, res, o_w, ln2_w.reshape(1, D), gate_pad)


def _k4_body(be_ref, xs_ref, eg_ref, eu_ref, ed_ref, out_ref):
    x = _unpack2(xs_ref[...])
    h1 = jnp.dot(x, eg_ref[0], preferred_element_type=jnp.float32)
    h2 = jnp.dot(x, eu_ref[0], preferred_element_type=jnp.float32)
    h = (h1 * jax.lax.logistic(h1)) * h2
    out_ref[...] = _pack2(jnp.dot(h, ed_ref[0],
                                  preferred_element_type=jnp.float32))


def _k4(xs, eg_w, eu_w, ed_w, block_expert):
    return pl.pallas_call(
        _k4_body,
        grid_spec=pltpu.PrefetchScalarGridSpec(
            num_scalar_prefetch=1,
            grid=(NB,),
            in_specs=[
                pl.BlockSpec((BLK, D // 2), lambda b, be: (b, 0)),
                pl.BlockSpec((1, D, DFF_MOE), lambda b, be: (be[b], 0, 0)),
                pl.BlockSpec((1, D, DFF_MOE), lambda b, be: (be[b], 0, 0)),
                pl.BlockSpec((1, DFF_MOE, D), lambda b, be: (be[b], 0, 0)),
            ],
            out_specs=pl.BlockSpec((BLK, D // 2), lambda b, be: (b, 0)),
        ),
        out_shape=jax.ShapeDtypeStruct((PN, D // 2), jnp.int32),
        compiler_params=pltpu.CompilerParams(
            vmem_limit_bytes=62 * 1024 * 1024),
    )(block_expert, xs, eg_w, eu_w, ed_w)


def _k5_body(n_ref, res2_ref, ea_ref, eb_ref, tw_ref,
             sg_ref, su_ref, sd_ref, out_ref):
    n = _unpack2(n_ref[...])
    g = jnp.dot(n, sg_ref[...], preferred_element_type=jnp.float32)
    u = jnp.dot(n, su_ref[...], preferred_element_type=jnp.float32)
    y = (g * jax.lax.logistic(g)) * u
    sh = jnp.dot(y, sd_ref[...], preferred_element_type=jnp.float32)
    routed = (tw_ref[:, 0:1] * _unpack2(ea_ref[...])
              + tw_ref[:, 1:2] * _unpack2(eb_ref[...]))
    out_ref[...] = res2_ref[...] + sh + routed


def _k5(normed, res2, ea, eb, twp, sg_w, su_w, sd_w):
    return pl.pallas_call(
        _k5_body,
        grid=(S // 128,),
        in_specs=[
            pl.BlockSpec((128, D // 2), lambda i: (i, 0)),
            pl.BlockSpec((128, D), lambda i: (i, 0)),
            pl.BlockSpec((128, D // 2), lambda i: (i, 0)),
            pl.BlockSpec((128, D // 2), lambda i: (i, 0)),
            pl.BlockSpec((128, 128), lambda i: (i, 0)),
            pl.BlockSpec((D, DFF_SH), lambda i: (0, 0)),
            pl.BlockSpec((D, DFF_SH), lambda i: (0, 0)),
            pl.BlockSpec((DFF_SH, D), lambda i: (0, 0)),
        ],
        out_specs=pl.BlockSpec((128, D), lambda i: (i, 0)),
        out_shape=jax.ShapeDtypeStruct((S, D), jnp.float32),
        compiler_params=pltpu.CompilerParams(
            vmem_limit_bytes=62 * 1024 * 1024),
    )(normed, res2, ea, eb, twp, sg_w, su_w, sd_w)


SC_NC, SC_NS = 2, 16
SC_NW = SC_NC * SC_NS


def _sc_gather_i32(table, idx, n_rows, chunk):
    b_per_w = n_rows // SC_NW
    nchunk = b_per_w // chunk
    ncols = table.shape[1]
    mesh = plsc.VectorSubcoreMesh(core_axis_name="c", subcore_axis_name="s")

    @functools.partial(
        pl.kernel, mesh=mesh,
        out_type=jax.ShapeDtypeStruct((n_rows, ncols), jnp.int32),
        scratch_types=[
            pltpu.VMEM((chunk,), jnp.int32),
            pltpu.VMEM((chunk, ncols), jnp.int32),
            pltpu.SemaphoreType.DMA,
        ],
    )
    def k(table_hbm, idx_hbm, out_hbm, idx_v, rows_v, sem):
        wid = lax.axis_index("s") * SC_NC + lax.axis_index("c")
        base = wid * b_per_w
        for c in range(nchunk):
            off = base + c * chunk
            pltpu.sync_copy(idx_hbm.at[pl.ds(off, chunk)], idx_v)
            pltpu.async_copy(table_hbm.at[idx_v], rows_v, sem).wait()
            pltpu.sync_copy(rows_v, out_hbm.at[pl.ds(off, chunk)])

    return k(table, idx)


def kernel(hidden_states, position_ids, cos, sin, ln1_w, q_w, k_w, v_w, o_w,
           qn_w, kn_w, ln2_w, gate_w, sg_w, su_w, sd_w, eg_w, eu_w, ed_w):
    hs = hidden_states.reshape(S, D)
    cos_s = cos[:S]
    sin_s = sin[:S]

    q, k, v = _k1(hs, cos_s, sin_s, ln1_w, q_w, k_w, v_w, qn_w, kn_w)
    attn = _k2(q, k, v)

    gate_pad = jnp.pad(gate_w, ((0, 0), (0, 128 - E)))
    res2, normed, comb = _k3(attn, hs, o_w, ln2_w, gate_pad)

    cw = comb[:, :E]
    tw2, ti2 = jax.lax.top_k(cw, TOPK)
    e_flat = ti2.reshape(-1)
    onehot = (e_flat[:, None] == jnp.arange(E)[None, :]).astype(jnp.int32)
    ranks = jnp.cumsum(onehot, axis=0) - onehot
    rank = jnp.take_along_axis(ranks, e_flat[:, None], axis=1)[:, 0]
    gsz = jnp.sum(onehot, axis=0)
    gpad = ((gsz + BLK - 1) // BLK) * BLK
    off = jnp.concatenate([jnp.zeros((1,), jnp.int32),
                           jnp.cumsum(gpad)]).astype(jnp.int32)
    pos = off[e_flat] + rank
    tokp = jnp.zeros((PN,), jnp.int32).at[pos].set(
        jnp.arange(2 * S, dtype=jnp.int32) // 2)
    block_expert = jnp.clip(
        jnp.searchsorted(off[:E], jnp.arange(NB, dtype=jnp.int32) * BLK,
                         side='right') - 1, 0, E - 1).astype(jnp.int32)

    xs = _sc_gather_i32(normed, tokp, PN, 80)
    eout = _k4(xs, eg_w, eu_w, ed_w, block_expert)

    posr = pos.reshape(S, 2)
    gidx = jnp.concatenate([posr[:, 0], posr[:, 1]])
    gath = _sc_gather_i32(eout, gidx, 2 * S, 64)
    ea = gath[:S]
    eb = gath[S:]
    twp = jnp.pad(tw2, ((0, 0), (0, 128 - TOPK)))

    out = _k5(normed, res2, ea, eb, twp, sg_w, su_w, sd_w)
    return out.reshape(B, S, D)
